# Initial kernel scaffold; baseline (speedup 1.0000x reference)
import jax
import jax.numpy as jnp
from jax.experimental import pallas as pl


def kernel(x):
    raise NotImplementedError("write your pallas kernel here")


# speedup vs baseline: 2.1271x; 2.1271x over previous
import functools

import numpy as np
import jax
import jax.numpy as jnp
from jax import lax
from jax.experimental import pallas as pl
from jax.experimental.pallas import tpu as pltpu
from jax.experimental.pallas import tpu_sc as plsc

_C, _H, _W = 3, 32, 32
_B, _D = 64, 256
_P = _C * _H * _W
_L = _P
_R = _L * _B


def _zigzag_gather_idx() -> np.ndarray:
    diagonals = [[] for _ in range(_H + _W - 1)]
    for i in range(_H):
        for j in range(_W):
            s = i + j
            if s % 2 == 0:
                diagonals[s].insert(0, (i, j))
            else:
                diagonals[s].append((i, j))
    pos = []
    for d in diagonals:
        for (i, j) in d:
            for c in range(_C):
                pos.append(c * _H * _W + i * _W + j)
    perm = np.asarray(pos, dtype=np.int64)
    r = np.arange(_R, dtype=np.int64)
    return ((r % _B) * _P + perm[r // _B]).astype(np.int32)


_GATHER_IDX = _zigzag_gather_idx()

_NW = 32
_ROWS_PER_W = _R // _NW
_K = 128
_CHUNKS = _ROWS_PER_W // _K


def _sc_gather(table, idx):
    mesh = plsc.VectorSubcoreMesh(core_axis_name="c", subcore_axis_name="s")

    @functools.partial(
        pl.kernel,
        mesh=mesh,
        out_type=jax.ShapeDtypeStruct((_R, _D), jnp.float32),
        scratch_types=[
            pltpu.VMEM((_K,), jnp.int32),
            pltpu.VMEM((_K, _D), jnp.float32),
            pltpu.SemaphoreType.DMA,
        ],
    )
    def k(table_hbm, idx_hbm, out_hbm, idx_v, rows_v, sem):
        wid = lax.axis_index("s") * 2 + lax.axis_index("c")
        base = wid * _ROWS_PER_W

        def body(i, carry):
            start = base + i * _K
            pltpu.sync_copy(idx_hbm.at[pl.ds(start, _K)], idx_v)
            pltpu.async_copy(table_hbm.at[idx_v], rows_v, sem).wait()
            pltpu.sync_copy(rows_v, out_hbm.at[pl.ds(start, _K)])
            return carry

        lax.fori_loop(0, _CHUNKS, body, 0)

    return k(table, idx)


def kernel(x):
    table = x.reshape(_B * _P, _D)
    out = _sc_gather(table, jnp.asarray(_GATHER_IDX))
    return out.reshape(_L, _B, _D)


# speedup vs baseline: 2.3773x; 1.1176x over previous
import functools

import numpy as np
import jax
import jax.numpy as jnp
from jax import lax
from jax.experimental import pallas as pl
from jax.experimental.pallas import tpu as pltpu
from jax.experimental.pallas import tpu_sc as plsc

_C, _H, _W = 3, 32, 32
_B, _D = 64, 256
_P = _C * _H * _W
_L = _P
_R = _L * _B


def _zigzag_gather_idx() -> np.ndarray:
    diagonals = [[] for _ in range(_H + _W - 1)]
    for i in range(_H):
        for j in range(_W):
            s = i + j
            if s % 2 == 0:
                diagonals[s].insert(0, (i, j))
            else:
                diagonals[s].append((i, j))
    pos = []
    for d in diagonals:
        for (i, j) in d:
            for c in range(_C):
                pos.append(c * _H * _W + i * _W + j)
    perm = np.asarray(pos, dtype=np.int64)
    r = np.arange(_R, dtype=np.int64)
    return ((r % _B) * _P + perm[r // _B]).astype(np.int32)


_GATHER_IDX = _zigzag_gather_idx()

_NW = 32
_ROWS_PER_W = _R // _NW
_K = 128
_CHUNKS = _ROWS_PER_W // _K


def _sc_gather(table, idx):
    mesh = plsc.VectorSubcoreMesh(core_axis_name="c", subcore_axis_name="s")

    @functools.partial(
        pl.kernel,
        mesh=mesh,
        out_type=jax.ShapeDtypeStruct((_R, _D), jnp.float32),
        scratch_types=[
            pltpu.VMEM((_ROWS_PER_W,), jnp.int32),
            pltpu.VMEM((_K, _D), jnp.float32),
            pltpu.SemaphoreType.DMA,
        ],
    )
    def k(table_hbm, idx_hbm, out_hbm, idx_v, rows_v, sem):
        wid = lax.axis_index("s") * 2 + lax.axis_index("c")
        base = wid * _ROWS_PER_W
        pltpu.sync_copy(idx_hbm.at[pl.ds(base, _ROWS_PER_W)], idx_v)

        def body(i, carry):
            start = base + i * _K
            pltpu.async_copy(
                table_hbm.at[idx_v.at[pl.ds(i * _K, _K)]], rows_v, sem
            ).wait()
            pltpu.sync_copy(rows_v, out_hbm.at[pl.ds(start, _K)])
            return carry

        lax.fori_loop(0, _CHUNKS, body, 0)

    return k(table, idx)


def kernel(x):
    table = x.reshape(_B * _P, _D)
    out = _sc_gather(table, jnp.asarray(_GATHER_IDX))
    return out.reshape(_L, _B, _D)


# speedup vs baseline: 2.8666x; 1.2058x over previous
import functools

import numpy as np
import jax
import jax.numpy as jnp
from jax import lax
from jax.experimental import pallas as pl
from jax.experimental.pallas import tpu as pltpu
from jax.experimental.pallas import tpu_sc as plsc

_C, _H, _W = 3, 32, 32
_B, _D = 64, 256
_P = _C * _H * _W
_L = _P
_R = _L * _B


def _zigzag_gather_idx() -> np.ndarray:
    diagonals = [[] for _ in range(_H + _W - 1)]
    for i in range(_H):
        for j in range(_W):
            s = i + j
            if s % 2 == 0:
                diagonals[s].insert(0, (i, j))
            else:
                diagonals[s].append((i, j))
    pos = []
    for d in diagonals:
        for (i, j) in d:
            for c in range(_C):
                pos.append(c * _H * _W + i * _W + j)
    perm = np.asarray(pos, dtype=np.int64)
    r = np.arange(_R, dtype=np.int64)
    return ((r % _B) * _P + perm[r // _B]).astype(np.int32)


_GATHER_IDX = _zigzag_gather_idx()

_NW = 32
_ROWS_PER_W = _R // _NW
_K = 128
_CHUNKS = _ROWS_PER_W // _K


def _sc_gather(table, idx):
    mesh = plsc.VectorSubcoreMesh(core_axis_name="c", subcore_axis_name="s")

    @functools.partial(
        pl.kernel,
        mesh=mesh,
        out_type=jax.ShapeDtypeStruct((_R, _D), jnp.float32),
        scratch_types=[
            pltpu.VMEM((_ROWS_PER_W,), jnp.int32),
            pltpu.VMEM((_K, _D), jnp.float32),
            pltpu.VMEM((_K, _D), jnp.float32),
            pltpu.SemaphoreType.DMA,
            pltpu.SemaphoreType.DMA,
            pltpu.SemaphoreType.DMA,
            pltpu.SemaphoreType.DMA,
        ],
    )
    def k(table_hbm, idx_hbm, out_hbm, idx_v, rows0_v, rows1_v,
          gsem0, gsem1, ssem0, ssem1):
        wid = lax.axis_index("s") * 2 + lax.axis_index("c")
        base = wid * _ROWS_PER_W
        pltpu.sync_copy(idx_hbm.at[pl.ds(base, _ROWS_PER_W)], idx_v)

        def gather(chunk, buf, sem):
            return pltpu.async_copy(
                table_hbm.at[idx_v.at[pl.ds(chunk * _K, _K)]], buf, sem
            )

        def gather_wait(chunk, buf, sem):
            pltpu.make_async_copy(
                table_hbm.at[idx_v.at[pl.ds(chunk * _K, _K)]], buf, sem
            ).wait()

        def scatter(chunk, buf, sem):
            return pltpu.async_copy(
                buf, out_hbm.at[pl.ds(base + chunk * _K, _K)], sem
            )

        def scatter_wait(chunk, buf, sem):
            pltpu.make_async_copy(
                buf, out_hbm.at[pl.ds(base + chunk * _K, _K)], sem
            ).wait()

        gather(0, rows0_v, gsem0)
        half = _CHUNKS // 2

        def body(t, carry):
            c0 = 2 * t
            c1 = c0 + 1

            @pl.when(t > 0)
            def _():
                scatter_wait(c0 - 1, rows1_v, ssem1)

            g1 = gather(c1, rows1_v, gsem1)
            gather_wait(c0, rows0_v, gsem0)
            s0 = scatter(c0, rows0_v, ssem0)
            g1.wait()
            s0.wait()

            @pl.when(t < half - 1)
            def _():
                gather(c1 + 1, rows0_v, gsem0)

            scatter(c1, rows1_v, ssem1)
            return carry

        lax.fori_loop(0, half, body, 0)
        scatter_wait(_CHUNKS - 1, rows1_v, ssem1)

    return k(table, idx)


def kernel(x):
    table = x.reshape(_B * _P, _D)
    out = _sc_gather(table, jnp.asarray(_GATHER_IDX))
    return out.reshape(_L, _B, _D)
